# Initial kernel scaffold; baseline (speedup 1.0000x reference)
import jax
import jax.numpy as jnp
from jax.experimental import pallas as pl


def kernel(scores):
    raise NotImplementedError("write your pallas kernel here")


# speedup vs baseline: 1195.0797x; 1195.0797x over previous
import jax
import jax.numpy as jnp
from jax.experimental import pallas as pl
from jax.experimental.pallas import tpu as pltpu

_TAU = 0.1
_EPS = 1e-10
_N = 4096
_BR = 512
_NB = _N // _BR


def _ccl_kernel(x_ref, out_ref, rowsum_ref, colsum_ref):
    p = pl.program_id(0)
    i = pl.program_id(1)
    e = jnp.exp(x_ref[...] * (1.0 / _TAU))

    @pl.when(p == 0)
    def _phase0():
        rowsum_ref[pl.ds(i * _BR, _BR), :] = jnp.sum(e, axis=1, keepdims=True)
        part = jnp.sum(e, axis=0, keepdims=True)

        @pl.when(i == 0)
        def _():
            colsum_ref[...] = part

        @pl.when(i > 0)
        def _():
            colsum_ref[...] = colsum_ref[...] + part

    @pl.when(p == 1)
    def _phase1():
        r = rowsum_ref[pl.ds(i * _BR, _BR), :] + _EPS
        c = colsum_ref[...] + _EPS
        a = 1.0 - e / r + _EPS
        b = 1.0 - e / c + _EPS
        rows = i * _BR + jax.lax.broadcasted_iota(jnp.int32, (_BR, _N), 0)
        cols = jax.lax.broadcasted_iota(jnp.int32, (_BR, _N), 1)
        prod = jnp.where(rows == cols, 1.0, a * b)
        s = jnp.sum(jnp.log(prod))

        @pl.when(i == 0)
        def _():
            out_ref[0, 0] = s

        @pl.when(i > 0)
        def _():
            out_ref[0, 0] = out_ref[0, 0] + s

        @pl.when(i == _NB - 1)
        def _():
            out_ref[0, 0] = out_ref[0, 0] * (-1.0 / _N)


def kernel(scores):
    out = pl.pallas_call(
        _ccl_kernel,
        grid=(2, _NB),
        in_specs=[pl.BlockSpec((_BR, _N), lambda p, i: (i, 0))],
        out_specs=pl.BlockSpec((1, 1), lambda p, i: (0, 0), memory_space=pltpu.SMEM),
        out_shape=jax.ShapeDtypeStruct((1, 1), jnp.float32),
        scratch_shapes=[
            pltpu.VMEM((_N, 1), jnp.float32),
            pltpu.VMEM((1, _N), jnp.float32),
        ],
    )(scores)
    return out[0, 0]


# speedup vs baseline: 1316.2695x; 1.1014x over previous
import jax
import jax.numpy as jnp
from jax.experimental import pallas as pl
from jax.experimental.pallas import tpu as pltpu

_TAU = 0.1
_EPS = 1e-10
_N = 4096
_BR = 512
_NB = _N // _BR


def _rowsum_mxu(m):
    ones = jnp.ones((m.shape[1], 1), dtype=jnp.float32)
    return jax.lax.dot_general(m, ones, (((1,), (0,)), ((), ())),
                               preferred_element_type=jnp.float32)


def _colsum_mxu(m):
    ones = jnp.ones((1, m.shape[0]), dtype=jnp.float32)
    return jax.lax.dot_general(ones, m, (((1,), (0,)), ((), ())),
                               preferred_element_type=jnp.float32)


def _ccl_kernel(x_ref, out_ref, rowsum_ref, colsum_ref):
    p = pl.program_id(0)
    i = pl.program_id(1)
    e = jnp.exp(x_ref[...] * (1.0 / _TAU))

    @pl.when(p == 0)
    def _phase0():
        rowsum_ref[pl.ds(i * _BR, _BR), :] = _rowsum_mxu(e)
        part = _colsum_mxu(e)

        @pl.when(i == 0)
        def _():
            colsum_ref[...] = part

        @pl.when(i > 0)
        def _():
            colsum_ref[...] = colsum_ref[...] + part

    @pl.when(p == 1)
    def _phase1():
        r = rowsum_ref[pl.ds(i * _BR, _BR), :] + _EPS
        c = colsum_ref[...] + _EPS
        a = 1.0 - e / r + _EPS
        b = 1.0 - e / c + _EPS
        t = jnp.log(a * b)
        s_full = jnp.sum(_colsum_mxu(t))

        e_d = jnp.exp(x_ref[:, pl.ds(i * _BR, _BR)] * (1.0 / _TAU))
        c_d = colsum_ref[:, pl.ds(i * _BR, _BR)] + _EPS
        a_d = 1.0 - e_d / r + _EPS
        b_d = 1.0 - e_d / c_d + _EPS
        t_d = jnp.log(a_d * b_d)
        rows = jax.lax.broadcasted_iota(jnp.int32, (_BR, _BR), 0)
        cols = jax.lax.broadcasted_iota(jnp.int32, (_BR, _BR), 1)
        diag_sum = jnp.sum(jnp.where(rows == cols, t_d, 0.0))

        s = s_full - diag_sum

        @pl.when(i == 0)
        def _():
            out_ref[0, 0] = s

        @pl.when(i > 0)
        def _():
            out_ref[0, 0] = out_ref[0, 0] + s

        @pl.when(i == _NB - 1)
        def _():
            out_ref[0, 0] = out_ref[0, 0] * (-1.0 / _N)


def kernel(scores):
    out = pl.pallas_call(
        _ccl_kernel,
        grid=(2, _NB),
        in_specs=[pl.BlockSpec((_BR, _N), lambda p, i: (i, 0))],
        out_specs=pl.BlockSpec((1, 1), lambda p, i: (0, 0), memory_space=pltpu.SMEM),
        out_shape=jax.ShapeDtypeStruct((1, 1), jnp.float32),
        scratch_shapes=[
            pltpu.VMEM((_N, 1), jnp.float32),
            pltpu.VMEM((1, _N), jnp.float32),
        ],
    )(scores)
    return out[0, 0]
